# Initial kernel scaffold; baseline (speedup 1.0000x reference)
import jax
import jax.numpy as jnp
from jax.experimental import pallas as pl


def kernel(xyz1, xyz2):
    raise NotImplementedError("write your pallas kernel here")


# speedup vs baseline: 1.2458x; 1.2458x over previous
import jax
import jax.numpy as jnp
from jax.experimental import pallas as pl
from jax.experimental.pallas import tpu as pltpu

_N = 2048
_M = 2048
_TR = 256
_TC = 256
_NITER = 10


def _emd_body(x1_ref, x2t_ref, out_ref, d_ref, e_ref, rl_ref, rr_ref, ratl_ref):
    x2t = x2t_ref[0]

    def dist_chunk(r, c0):
        xr = x1_ref[0, pl.ds(r * _TR, _TR), :]
        acc = jnp.zeros((_TR, _M), jnp.float32)
        for c in range(3):
            diff = xr[:, c : c + 1] - x2t[c : c + 1, :]
            acc = acc + diff * diff
        d_ref[pl.ds(r * _TR, _TR), :] = jnp.sqrt(acc)
        return c0

    jax.lax.fori_loop(0, _N // _TR, dist_chunk, 0)

    rl_ref[:] = jnp.ones((_N, 1), jnp.float32)
    rr_ref[:] = jnp.ones((1, _M), jnp.float32)

    def iter_body(i, cost):
        fi = i.astype(jnp.float32)
        level = jnp.where(i == _NITER - 1, 0.0, -jnp.exp2(2.0 * (7.0 - fi)))

        def p1(r, c0):
            rows = pl.ds(r * _TR, _TR)
            dch = d_ref[rows, :]
            e = jnp.exp(level * dch * dch)
            e_ref[rows, :] = e
            suml = 1e-9 + jnp.sum(e * rr_ref[:], axis=1, keepdims=True)
            ratl_ref[rows, :] = rl_ref[rows, :] / suml
            return c0

        jax.lax.fori_loop(0, _N // _TR, p1, 0)

        ratioL = ratl_ref[:]

        def p23(c, carry):
            rowacc, costrow = carry
            cols = pl.ds(c * _TC, _TC)
            e = e_ref[:, cols]
            sumr = jnp.sum(e * ratioL, axis=0, keepdims=True)
            rrc = rr_ref[0:1, cols]
            sumr = sumr * rrc
            cons = jnp.minimum(rrc / (sumr + 1e-9), 1.0)
            ratioR = cons * rrc
            rr_ref[0:1, cols] = jnp.maximum(0.0, rrc - sumr)
            w = e * ratioR
            rowacc = rowacc + jnp.sum(w, axis=1, keepdims=True)
            costrow = costrow + jnp.sum(d_ref[:, cols] * w, axis=1, keepdims=True)
            return rowacc, costrow

        z = jnp.zeros((_N, 1), jnp.float32)
        rowacc, costrow = jax.lax.fori_loop(0, _M // _TC, p23, (z, z))
        rl_ref[:] = jnp.maximum(0.0, rl_ref[:] - ratioL * rowacc)
        return cost + jnp.sum(ratioL * costrow, keepdims=True)

    cost = jax.lax.fori_loop(0, _NITER, iter_body, jnp.zeros((1, 1), jnp.float32))
    out_ref[0] = cost


def kernel(xyz1, xyz2):
    b = xyz1.shape[0]
    x2t = jnp.transpose(xyz2, (0, 2, 1))
    costs = pl.pallas_call(
        _emd_body,
        grid=(b,),
        in_specs=[
            pl.BlockSpec((1, _N, 3), lambda i: (i, 0, 0)),
            pl.BlockSpec((1, 3, _M), lambda i: (i, 0, 0)),
        ],
        out_specs=pl.BlockSpec((1, 1, 1), lambda i: (i, 0, 0)),
        out_shape=jax.ShapeDtypeStruct((b, 1, 1), jnp.float32),
        scratch_shapes=[
            pltpu.VMEM((_N, _M), jnp.float32),
            pltpu.VMEM((_N, _M), jnp.float32),
            pltpu.VMEM((_N, 1), jnp.float32),
            pltpu.VMEM((1, _M), jnp.float32),
            pltpu.VMEM((_N, 1), jnp.float32),
        ],
        compiler_params=pltpu.CompilerParams(dimension_semantics=("parallel",)),
    )(xyz1, x2t)
    return jnp.mean(costs)


# speedup vs baseline: 2.3970x; 1.9241x over previous
import jax
import jax.numpy as jnp
from jax.experimental import pallas as pl
from jax.experimental.pallas import tpu as pltpu

_N = 2048
_M = 2048
_TL = 256
_NITER = 10
_LOG2E = 1.4426950408889634


def _level2(i):
    return 0.0 if i == _NITER - 1 else -(4.0 ** (7 - i)) * _LOG2E


def _emd_body(x2_ref, x1t_ref, out_ref, d_ref, e_ref, rr_ref):
    x1t = x1t_ref[0]

    def pro(r, suml):
        rows = pl.ds(r * _TL, _TL)
        xr = x2_ref[0, rows, :]
        acc = jnp.zeros((_TL, _N), jnp.float32)
        for c in range(3):
            diff = xr[:, c : c + 1] - x1t[c : c + 1, :]
            acc = acc + diff * diff
        d_ref[rows, :] = jnp.sqrt(acc)
        e = jnp.exp2(_level2(0) * acc)
        e_ref[rows, :] = e
        return suml + jnp.sum(e, axis=0, keepdims=True)

    suml = jax.lax.fori_loop(0, _M // _TL, pro, jnp.zeros((1, _N), jnp.float32))

    rr_ref[:] = jnp.ones((_M, 1), jnp.float32)
    remainL = jnp.ones((1, _N), jnp.float32)
    cost = jnp.zeros((1, 1), jnp.float32)

    for i in range(_NITER):
        ratioL = remainL / (1e-9 + suml)
        last = i == _NITER - 1
        lvl2n = _level2(i + 1) if not last else 0.0

        def body(r, carry, ratioL=ratioL, lvl2n=lvl2n, last=last):
            rowacc, costrow, sumln = carry
            rows = pl.ds(r * _TL, _TL)
            e = e_ref[rows, :]
            dch = d_ref[rows, :]
            sumr = jnp.sum(e * ratioL, axis=1, keepdims=True)
            rrc = rr_ref[rows, :]
            sumr = sumr * rrc
            cons = jnp.minimum(rrc / (sumr + 1e-9), 1.0)
            ratioR = cons * rrc
            rrn = jnp.maximum(0.0, rrc - sumr)
            rr_ref[rows, :] = rrn
            w = e * ratioR
            rowacc = rowacc + jnp.sum(w, axis=0, keepdims=True)
            costrow = costrow + jnp.sum(dch * w, axis=0, keepdims=True)
            if not last:
                en = jnp.exp2(lvl2n * (dch * dch))
                e_ref[rows, :] = en
                sumln = sumln + jnp.sum(en * rrn, axis=0, keepdims=True)
            return rowacc, costrow, sumln

        z = jnp.zeros((1, _N), jnp.float32)
        rowacc, costrow, suml = jax.lax.fori_loop(0, _M // _TL, body, (z, z, z))
        remainL = jnp.maximum(0.0, remainL - ratioL * rowacc)
        cost = cost + jnp.sum(ratioL * costrow, keepdims=True)

    out_ref[0] = cost


def kernel(xyz1, xyz2):
    b = xyz1.shape[0]
    x1t = jnp.transpose(xyz1, (0, 2, 1))
    costs = pl.pallas_call(
        _emd_body,
        grid=(b,),
        in_specs=[
            pl.BlockSpec((1, _M, 3), lambda i: (i, 0, 0)),
            pl.BlockSpec((1, 3, _N), lambda i: (i, 0, 0)),
        ],
        out_specs=pl.BlockSpec((1, 1, 1), lambda i: (i, 0, 0)),
        out_shape=jax.ShapeDtypeStruct((b, 1, 1), jnp.float32),
        scratch_shapes=[
            pltpu.VMEM((_M, _N), jnp.float32),
            pltpu.VMEM((_M, _N), jnp.float32),
            pltpu.VMEM((_M, 1), jnp.float32),
        ],
        compiler_params=pltpu.CompilerParams(dimension_semantics=("parallel",)),
    )(xyz2, x1t)
    return jnp.mean(costs)


# speedup vs baseline: 2.5341x; 1.0572x over previous
import jax
import jax.numpy as jnp
from jax.experimental import pallas as pl
from jax.experimental.pallas import tpu as pltpu

_N = 2048
_M = 2048
_TL = 512
_NITER = 10
_LOG2E = 1.4426950408889634


def _level2(i):
    return 0.0 if i == _NITER - 1 else -(4.0 ** (7 - i)) * _LOG2E


def _emd_body(x2_ref, x1t_ref, out_ref, d_ref, e_ref, rr_ref):
    x1t = x1t_ref[0]

    def pro(r, suml):
        rows = pl.ds(r * _TL, _TL)
        xr = x2_ref[0, rows, :]
        acc = jnp.zeros((_TL, _N), jnp.float32)
        for c in range(3):
            diff = xr[:, c : c + 1] - x1t[c : c + 1, :]
            acc = acc + diff * diff
        d_ref[rows, :] = jnp.sqrt(acc)
        e = jnp.exp2(_level2(0) * acc)
        e_ref[rows, :] = e
        return suml + jnp.sum(e, axis=0, keepdims=True)

    suml = jax.lax.fori_loop(0, _M // _TL, pro, jnp.zeros((1, _N), jnp.float32))

    rr_ref[:] = jnp.ones((_M, 1), jnp.float32)
    remainL = jnp.ones((1, _N), jnp.float32)
    cost = jnp.zeros((1, 1), jnp.float32)

    for i in range(_NITER):
        ratioL = remainL / (1e-9 + suml)
        last = i == _NITER - 1
        lvl2n = _level2(i + 1) if not last else 0.0

        def body(r, carry, ratioL=ratioL, lvl2n=lvl2n, last=last):
            rowacc, costrow, sumln = carry
            rows = pl.ds(r * _TL, _TL)
            e = e_ref[rows, :]
            dch = d_ref[rows, :]
            sumr = jax.lax.dot_general(
                e, ratioL, (((1,), (1,)), ((), ())),
                preferred_element_type=jnp.float32)
            rrc = rr_ref[rows, :]
            sumr = sumr * rrc
            cons = jnp.minimum(rrc / (sumr + 1e-9), 1.0)
            ratioR = cons * rrc
            rrn = jnp.maximum(0.0, rrc - sumr)
            rr_ref[rows, :] = rrn
            w = e * ratioR
            rowacc = rowacc + jnp.sum(w, axis=0, keepdims=True)
            costrow = costrow + jnp.sum(dch * w, axis=0, keepdims=True)
            if not last:
                en = jnp.exp2(lvl2n * (dch * dch))
                e_ref[rows, :] = en
                sumln = sumln + jnp.sum(en * rrn, axis=0, keepdims=True)
            return rowacc, costrow, sumln

        z = jnp.zeros((1, _N), jnp.float32)
        rowacc, costrow, suml = jax.lax.fori_loop(0, _M // _TL, body, (z, z, z))
        remainL = jnp.maximum(0.0, remainL - ratioL * rowacc)
        cost = cost + jnp.sum(ratioL * costrow, keepdims=True)

    out_ref[0] = cost


def kernel(xyz1, xyz2):
    b = xyz1.shape[0]
    x1t = jnp.transpose(xyz1, (0, 2, 1))
    costs = pl.pallas_call(
        _emd_body,
        grid=(b,),
        in_specs=[
            pl.BlockSpec((1, _M, 3), lambda i: (i, 0, 0)),
            pl.BlockSpec((1, 3, _N), lambda i: (i, 0, 0)),
        ],
        out_specs=pl.BlockSpec((1, 1, 1), lambda i: (i, 0, 0)),
        out_shape=jax.ShapeDtypeStruct((b, 1, 1), jnp.float32),
        scratch_shapes=[
            pltpu.VMEM((_M, _N), jnp.float32),
            pltpu.VMEM((_M, _N), jnp.float32),
            pltpu.VMEM((_M, 1), jnp.float32),
        ],
        compiler_params=pltpu.CompilerParams(dimension_semantics=("parallel",)),
    )(xyz2, x1t)
    return jnp.mean(costs)


# speedup vs baseline: 2.6695x; 1.0534x over previous
import jax
import jax.numpy as jnp
from jax.experimental import pallas as pl
from jax.experimental.pallas import tpu as pltpu

_N = 2048
_M = 2048
_TL = 512
_NITER = 10
_LOG2E = 1.4426950408889634


def _level2(i):
    return 0.0 if i == _NITER - 1 else -(4.0 ** (7 - i)) * _LOG2E


def _emd_body(x2_ref, x1t_ref, out_ref, d_ref, e_ref, rr_ref):
    x1t = x1t_ref[0]

    def pro(r, suml):
        rows = pl.ds(r * _TL, _TL)
        xr = x2_ref[0, rows, :]
        acc = jnp.zeros((_TL, _N), jnp.float32)
        for c in range(3):
            diff = xr[:, c : c + 1] - x1t[c : c + 1, :]
            acc = acc + diff * diff
        d_ref[rows, :] = jnp.sqrt(acc)
        e = jnp.exp2(_level2(0) * acc)
        e_ref[rows, :] = e
        return suml + jnp.sum(e, axis=0, keepdims=True)

    suml = jax.lax.fori_loop(0, _M // _TL, pro, jnp.zeros((1, _N), jnp.float32))

    rr_ref[:] = jnp.ones((_M, 1), jnp.float32)
    remainL = jnp.ones((1, _N), jnp.float32)
    cost = jnp.zeros((1, 1), jnp.float32)

    for i in range(_NITER - 1):
        ratioL = remainL / (1e-9 + suml)
        last = i == _NITER - 2
        lvl2n = _level2(i + 1) if not last else 0.0

        def body(r, carry, ratioL=ratioL, lvl2n=lvl2n, last=last):
            rowacc, costrow, sumln = carry
            rows = pl.ds(r * _TL, _TL)
            e = e_ref[rows, :]
            dch = d_ref[rows, :]
            sumr = jax.lax.dot_general(
                e, ratioL, (((1,), (1,)), ((), ())),
                preferred_element_type=jnp.float32)
            rrc = rr_ref[rows, :]
            sumr = sumr * rrc
            cons = jnp.minimum(rrc / (sumr + 1e-9), 1.0)
            ratioR = cons * rrc
            rrn = jnp.maximum(0.0, rrc - sumr)
            rr_ref[rows, :] = rrn
            w = e * ratioR
            rowacc = rowacc + jnp.sum(w, axis=0, keepdims=True)
            costrow = costrow + jnp.sum(dch * w, axis=0, keepdims=True)
            if not last:
                en = jnp.exp2(lvl2n * (dch * dch))
                e_ref[rows, :] = en
                sumln = sumln + jnp.sum(en * rrn, axis=0, keepdims=True)
            return rowacc, costrow, sumln

        z = jnp.zeros((1, _N), jnp.float32)
        rowacc, costrow, suml = jax.lax.fori_loop(0, _M // _TL, body, (z, z, z))
        remainL = jnp.maximum(0.0, remainL - ratioL * rowacc)
        cost = cost + jnp.sum(ratioL * costrow, keepdims=True)

    s_rem = jnp.sum(rr_ref[:], keepdims=True)
    ratioL = remainL / (1e-9 + s_rem)
    s_ratl = jnp.sum(ratioL, keepdims=True)
    rrc = rr_ref[:]
    sumr = s_ratl * rrc
    cons = jnp.minimum(rrc / (sumr + 1e-9), 1.0)
    ratioR_ref = rr_ref
    ratioR_ref[:] = cons * rrc

    def tail(r, costrow):
        rows = pl.ds(r * _TL, _TL)
        dch = d_ref[rows, :]
        return costrow + jnp.sum(dch * ratioR_ref[rows, :], axis=0, keepdims=True)

    costrow = jax.lax.fori_loop(
        0, _M // _TL, tail, jnp.zeros((1, _N), jnp.float32))
    cost = cost + jnp.sum(ratioL * costrow, keepdims=True)

    out_ref[0] = cost


def kernel(xyz1, xyz2):
    b = xyz1.shape[0]
    x1t = jnp.transpose(xyz1, (0, 2, 1))
    costs = pl.pallas_call(
        _emd_body,
        grid=(b,),
        in_specs=[
            pl.BlockSpec((1, _M, 3), lambda i: (i, 0, 0)),
            pl.BlockSpec((1, 3, _N), lambda i: (i, 0, 0)),
        ],
        out_specs=pl.BlockSpec((1, 1, 1), lambda i: (i, 0, 0)),
        out_shape=jax.ShapeDtypeStruct((b, 1, 1), jnp.float32),
        scratch_shapes=[
            pltpu.VMEM((_M, _N), jnp.float32),
            pltpu.VMEM((_M, _N), jnp.float32),
            pltpu.VMEM((_M, 1), jnp.float32),
        ],
        compiler_params=pltpu.CompilerParams(dimension_semantics=("parallel",)),
    )(xyz2, x1t)
    return jnp.mean(costs)


# speedup vs baseline: 2.8995x; 1.0862x over previous
import jax
import jax.numpy as jnp
from jax.experimental import pallas as pl
from jax.experimental.pallas import tpu as pltpu

_N = 2048
_M = 2048
_TL = 512
_NITER = 10
_LOG2E = 1.4426950408889634


def _level2(i):
    return 0.0 if i == _NITER - 1 else -(4.0 ** (7 - i)) * _LOG2E


def _emd_body(x2_ref, x1t_ref, out_ref, d_ref, e_ref, rr_ref):
    x1t = x1t_ref[0]

    def pro(r, suml):
        rows = pl.ds(r * _TL, _TL)
        xr = x2_ref[0, rows, :]
        acc = jnp.zeros((_TL, _N), jnp.float32)
        for c in range(3):
            diff = xr[:, c : c + 1] - x1t[c : c + 1, :]
            acc = acc + diff * diff
        d_ref[rows, :] = jnp.sqrt(acc)
        e = jnp.exp2(_level2(0) * acc)
        e_ref[rows, :] = e
        return suml + jnp.sum(e, axis=0, keepdims=True)

    suml = jax.lax.fori_loop(0, _M // _TL, pro, jnp.zeros((1, _N), jnp.float32))

    rr_ref[:] = jnp.ones((_M, 1), jnp.float32)
    remainL = jnp.ones((1, _N), jnp.float32)
    cost = jnp.zeros((1, 1), jnp.float32)

    for i in range(_NITER - 1):
        ratioL = remainL / (1e-9 + suml)
        last = i == _NITER - 2
        lvl2n = _level2(i + 1) if not last else 0.0

        def body(r, carry, ratioL=ratioL, lvl2n=lvl2n, last=last):
            rowacc, costrow, sumln = carry
            rows = pl.ds(r * _TL, _TL)
            e = e_ref[rows, :]
            dch = d_ref[rows, :]
            sumr = jax.lax.dot_general(
                e, ratioL, (((1,), (1,)), ((), ())),
                preferred_element_type=jnp.float32)
            rrc = rr_ref[rows, :]
            sumr = sumr * rrc
            cons = jnp.minimum(rrc / (sumr + 1e-9), 1.0)
            ratioR = cons * rrc
            rrn = jnp.maximum(0.0, rrc - sumr)
            rr_ref[rows, :] = rrn
            de = dch * e
            rowacc = rowacc + jax.lax.dot_general(
                ratioR, e, (((0,), (0,)), ((), ())),
                preferred_element_type=jnp.float32)
            costrow = costrow + jax.lax.dot_general(
                ratioR, de, (((0,), (0,)), ((), ())),
                preferred_element_type=jnp.float32)
            if not last:
                en = jnp.exp2(lvl2n * (dch * dch))
                e_ref[rows, :] = en
                sumln = sumln + jax.lax.dot_general(
                    rrn, en, (((0,), (0,)), ((), ())),
                    preferred_element_type=jnp.float32)
            return rowacc, costrow, sumln

        z = jnp.zeros((1, _N), jnp.float32)
        rowacc, costrow, suml = jax.lax.fori_loop(0, _M // _TL, body, (z, z, z))
        remainL = jnp.maximum(0.0, remainL - ratioL * rowacc)
        cost = cost + jnp.sum(ratioL * costrow, keepdims=True)

    s_rem = jnp.sum(rr_ref[:], keepdims=True)
    ratioL = remainL / (1e-9 + s_rem)
    s_ratl = jnp.sum(ratioL, keepdims=True)
    rrc = rr_ref[:]
    sumr = s_ratl * rrc
    cons = jnp.minimum(rrc / (sumr + 1e-9), 1.0)
    ratioR_ref = rr_ref
    ratioR_ref[:] = cons * rrc

    def tail(r, costrow):
        rows = pl.ds(r * _TL, _TL)
        dch = d_ref[rows, :]
        return costrow + jnp.sum(dch * ratioR_ref[rows, :], axis=0, keepdims=True)

    costrow = jax.lax.fori_loop(
        0, _M // _TL, tail, jnp.zeros((1, _N), jnp.float32))
    cost = cost + jnp.sum(ratioL * costrow, keepdims=True)

    out_ref[0] = cost


def kernel(xyz1, xyz2):
    b = xyz1.shape[0]
    x1t = jnp.transpose(xyz1, (0, 2, 1))
    costs = pl.pallas_call(
        _emd_body,
        grid=(b,),
        in_specs=[
            pl.BlockSpec((1, _M, 3), lambda i: (i, 0, 0)),
            pl.BlockSpec((1, 3, _N), lambda i: (i, 0, 0)),
        ],
        out_specs=pl.BlockSpec((1, 1, 1), lambda i: (i, 0, 0)),
        out_shape=jax.ShapeDtypeStruct((b, 1, 1), jnp.float32),
        scratch_shapes=[
            pltpu.VMEM((_M, _N), jnp.float32),
            pltpu.VMEM((_M, _N), jnp.float32),
            pltpu.VMEM((_M, 1), jnp.float32),
        ],
        compiler_params=pltpu.CompilerParams(dimension_semantics=("parallel",)),
    )(xyz2, x1t)
    return jnp.mean(costs)
